# baseline (speedup 1.0000x reference)
import functools

import jax
import jax.numpy as jnp
from jax.experimental import pallas as pl


def _mm_kernel(x_ref, w_ref, b_ref, o_ref, *, act):
    o = jnp.dot(x_ref[...], w_ref[...], preferred_element_type=jnp.float32)
    o = o + b_ref[...]
    if act:
        o = jnp.maximum(o, 0.0)
    o_ref[...] = o


def _mm(x, w, b, act=False, tile=1000):
    n, k = x.shape
    m = w.shape[1]
    pad = (-n) % tile
    if pad:
        x = jnp.pad(x, ((0, pad), (0, 0)))
    npad = n + pad
    out = pl.pallas_call(
        functools.partial(_mm_kernel, act=act),
        grid=(npad // tile,),
        in_specs=[
            pl.BlockSpec((tile, k), lambda i: (i, 0)),
            pl.BlockSpec((k, m), lambda i: (0, 0)),
            pl.BlockSpec((1, m), lambda i: (0, 0)),
        ],
        out_specs=pl.BlockSpec((tile, m), lambda i: (i, 0)),
        out_shape=jax.ShapeDtypeStruct((npad, m), jnp.float32),
    )(x, w, b.reshape(1, m))
    return out[:n]


def _feat_kernel(xw_ref, ww_ref, wfc_ref, bfc_ref, o_ref):
    xw = xw_ref[...]
    ww = ww_ref[...]
    y0 = (xw * ww[0][None, None, :]).sum(-1)
    y1 = (xw * ww[1][None, None, :]).sum(-1)
    y2 = (xw * ww[2][None, None, :]).sum(-1)
    pooled = jnp.maximum(jnp.maximum(y0, y1), y2)
    o_ref[...] = (
        jnp.dot(pooled, wfc_ref[...], preferred_element_type=jnp.float32)
        + bfc_ref[...]
    )


def _fuse_kernel(a_ref, b_ref, w_ref, bias_ref, o_ref):
    prod = a_ref[...] * b_ref[...]
    o_ref[...] = (
        jnp.dot(prod, w_ref[...], preferred_element_type=jnp.float32)
        + bias_ref[...]
    )


def _gat_aggregate(xp, asrc, adst, s, d, n):
    alpha = jax.nn.leaky_relu(asrc[s] + adst[d], 0.2)
    amax = jax.ops.segment_max(alpha, d, num_segments=n)
    e = jnp.exp(alpha - amax[d])
    denom = jax.ops.segment_sum(e, d, num_segments=n)
    coef = e / (denom[d] + 1e-16)
    msgs = xp[s] * coef[:, :, None]
    return jax.ops.segment_sum(msgs, d, num_segments=n)


def kernel(x, edge_index, train_edge_id, W_conv, b_conv, bn_g, bn_b, bn_rm,
           bn_rv, W_fc1, b_fc1, W_g1, a1_s, a1_d, b_g1, W_g2, a2_s, a2_d,
           b_g2, W_l1, b_l1, W_l2, b_l2, W_fc2, b_fc2):
    n = x.shape[0]
    lp = (x.shape[1] - 2) // 3
    gin = W_fc1.shape[1]

    inv = 1.0 / jnp.sqrt(bn_rv[0] + 1e-5)
    scale = bn_g[0] * inv
    cb = (b_conv[0] - bn_rm[0]) * scale + bn_b[0]
    wt = (W_conv[0] * scale).T
    ww = jnp.zeros((3, 5, 13), jnp.float32)
    for j in range(3):
        ww = ww.at[j, j:j + 3, :].set(wt)
    ww = ww.reshape(3, 65)
    bfc1 = b_fc1 + cb * W_fc1.sum(axis=0)

    xs = [x[:, t:t + 498:3, :] for t in range(5)]
    xwin = jnp.stack(xs, axis=2).reshape(n, lp, 65)

    tile = 80
    y = pl.pallas_call(
        _feat_kernel,
        grid=(n // tile,),
        in_specs=[
            pl.BlockSpec((tile, lp, 65), lambda i: (i, 0, 0)),
            pl.BlockSpec((3, 65), lambda i: (0, 0)),
            pl.BlockSpec((lp, gin), lambda i: (0, 0)),
            pl.BlockSpec((1, gin), lambda i: (0, 0)),
        ],
        out_specs=pl.BlockSpec((tile, gin), lambda i: (i, 0)),
        out_shape=jax.ShapeDtypeStruct((n, gin), jnp.float32),
    )(xwin, ww, W_fc1, bfc1.reshape(1, gin))

    src, dst = edge_index[0], edge_index[1]
    loops = jnp.arange(n, dtype=src.dtype)
    s = jnp.concatenate([src, loops])
    d = jnp.concatenate([dst, loops])

    h1dim = W_g1.shape[1]
    heads1, ch1 = a1_s.shape
    blk_s = (jnp.eye(heads1)[:, None, :] * a1_s[:, :, None]).reshape(h1dim, heads1)
    blk_d = (jnp.eye(heads1)[:, None, :] * a1_d[:, :, None]).reshape(h1dim, heads1)
    wcat1 = jnp.concatenate([W_g1, W_g1 @ blk_s, W_g1 @ blk_d], axis=1)
    f1 = _mm(y, wcat1, jnp.zeros((wcat1.shape[1],), jnp.float32))
    xp1 = f1[:, :h1dim].reshape(n, heads1, ch1)
    out1 = _gat_aggregate(xp1, f1[:, h1dim:h1dim + heads1],
                          f1[:, h1dim + heads1:], s, d, n)
    h1 = jnp.maximum(out1.reshape(n, h1dim) + b_g1, 0.0)

    h2dim = W_g2.shape[1]
    wcat2 = jnp.concatenate([W_g2, W_g2 @ a2_s[0][:, None],
                             W_g2 @ a2_d[0][:, None]], axis=1)
    f2 = _mm(h1, wcat2, jnp.zeros((wcat2.shape[1],), jnp.float32))
    xp2 = f2[:, :h2dim].reshape(n, 1, h2dim)
    out2 = _gat_aggregate(xp2, f2[:, h2dim:h2dim + 1],
                          f2[:, h2dim + 1:], s, d, n)
    h2 = out2.reshape(n, h2dim) + b_g2

    t1 = _mm(h2, W_l1, b_l1, act=True)
    h = _mm(t1, W_l2, b_l2)

    node_id = edge_index[:, train_edge_id]
    ha = h[node_id[0]]
    hb = h[node_id[1]]
    ne = ha.shape[0]
    cls = W_fc2.shape[1]
    tile_e = 1024
    out = pl.pallas_call(
        _fuse_kernel,
        grid=(ne // tile_e,),
        in_specs=[
            pl.BlockSpec((tile_e, h2dim), lambda i: (i, 0)),
            pl.BlockSpec((tile_e, h2dim), lambda i: (i, 0)),
            pl.BlockSpec((h2dim, cls), lambda i: (0, 0)),
            pl.BlockSpec((1, cls), lambda i: (0, 0)),
        ],
        out_specs=pl.BlockSpec((tile_e, cls), lambda i: (i, 0)),
        out_shape=jax.ShapeDtypeStruct((ne, cls), jnp.float32),
    )(ha, hb, W_fc2, b_fc2.reshape(1, cls))
    return out


# speedup vs baseline: 1.0009x; 1.0009x over previous
import functools

import jax
import jax.numpy as jnp
from jax.experimental import pallas as pl


def _mm_kernel(x_ref, w_ref, b_ref, o_ref, *, act):
    o = jnp.dot(x_ref[...], w_ref[...], preferred_element_type=jnp.float32)
    o = o + b_ref[...]
    if act:
        o = jnp.maximum(o, 0.0)
    o_ref[...] = o


def _mm(x, w, b, act=False, tile=1000):
    n, k = x.shape
    m = w.shape[1]
    pad = (-n) % tile
    if pad:
        x = jnp.pad(x, ((0, pad), (0, 0)))
    npad = n + pad
    out = pl.pallas_call(
        functools.partial(_mm_kernel, act=act),
        grid=(npad // tile,),
        in_specs=[
            pl.BlockSpec((tile, k), lambda i: (i, 0)),
            pl.BlockSpec((k, m), lambda i: (0, 0)),
            pl.BlockSpec((1, m), lambda i: (0, 0)),
        ],
        out_specs=pl.BlockSpec((tile, m), lambda i: (i, 0)),
        out_shape=jax.ShapeDtypeStruct((npad, m), jnp.float32),
    )(x, w, b.reshape(1, m))
    return out[:n]


def _feat_kernel(xw_ref, ww_ref, wfc_ref, bfc_ref, o_ref):
    xw = xw_ref[...]
    ww = ww_ref[...]
    y0 = (xw * ww[0][None, None, :]).sum(-1)
    y1 = (xw * ww[1][None, None, :]).sum(-1)
    y2 = (xw * ww[2][None, None, :]).sum(-1)
    pooled = jnp.maximum(jnp.maximum(y0, y1), y2)
    o_ref[...] = (
        jnp.dot(pooled, wfc_ref[...], preferred_element_type=jnp.float32)
        + bfc_ref[...]
    )


def _mlp_kernel(x_ref, w1_ref, b1_ref, w2_ref, b2_ref, o_ref):
    t = jnp.maximum(
        jnp.dot(x_ref[...], w1_ref[...], preferred_element_type=jnp.float32)
        + b1_ref[...], 0.0)
    o_ref[...] = (
        jnp.dot(t, w2_ref[...], preferred_element_type=jnp.float32)
        + b2_ref[...]
    )


def _fuse_kernel(a_ref, b_ref, w_ref, bias_ref, o_ref):
    prod = a_ref[...] * b_ref[...]
    o_ref[...] = (
        jnp.dot(prod, w_ref[...], preferred_element_type=jnp.float32)
        + bias_ref[...]
    )


def _gat_aggregate(xp, asrc, adst, s, d, n):
    alpha = jax.nn.leaky_relu(asrc[s] + adst[d], 0.2)
    amax = jax.ops.segment_max(alpha, d, num_segments=n)
    e = jnp.exp(alpha - amax[d])
    denom = jax.ops.segment_sum(e, d, num_segments=n)
    coef = e / (denom[d] + 1e-16)
    msgs = xp[s] * coef[:, :, None]
    return jax.ops.segment_sum(msgs, d, num_segments=n)


def kernel(x, edge_index, train_edge_id, W_conv, b_conv, bn_g, bn_b, bn_rm,
           bn_rv, W_fc1, b_fc1, W_g1, a1_s, a1_d, b_g1, W_g2, a2_s, a2_d,
           b_g2, W_l1, b_l1, W_l2, b_l2, W_fc2, b_fc2):
    n = x.shape[0]
    lp = (x.shape[1] - 2) // 3
    gin = W_fc1.shape[1]

    inv = 1.0 / jnp.sqrt(bn_rv[0] + 1e-5)
    scale = bn_g[0] * inv
    cb = (b_conv[0] - bn_rm[0]) * scale + bn_b[0]
    wt = (W_conv[0] * scale).T
    ww = jnp.zeros((3, 5, 13), jnp.float32)
    for j in range(3):
        ww = ww.at[j, j:j + 3, :].set(wt)
    ww = ww.reshape(3, 65)
    bfc1 = b_fc1 + cb * W_fc1.sum(axis=0)

    xs = [x[:, t:t + 498:3, :] for t in range(5)]
    xwin = jnp.stack(xs, axis=2).reshape(n, lp, 65)

    tile = 80
    y = pl.pallas_call(
        _feat_kernel,
        grid=(n // tile,),
        in_specs=[
            pl.BlockSpec((tile, lp, 65), lambda i: (i, 0, 0)),
            pl.BlockSpec((3, 65), lambda i: (0, 0)),
            pl.BlockSpec((lp, gin), lambda i: (0, 0)),
            pl.BlockSpec((1, gin), lambda i: (0, 0)),
        ],
        out_specs=pl.BlockSpec((tile, gin), lambda i: (i, 0)),
        out_shape=jax.ShapeDtypeStruct((n, gin), jnp.float32),
    )(xwin, ww, W_fc1, bfc1.reshape(1, gin))

    src, dst = edge_index[0], edge_index[1]
    loops = jnp.arange(n, dtype=src.dtype)
    s = jnp.concatenate([src, loops])
    d = jnp.concatenate([dst, loops])

    h1dim = W_g1.shape[1]
    heads1, ch1 = a1_s.shape
    blk_s = (jnp.eye(heads1)[:, None, :] * a1_s[:, :, None]).reshape(h1dim, heads1)
    blk_d = (jnp.eye(heads1)[:, None, :] * a1_d[:, :, None]).reshape(h1dim, heads1)
    wcat1 = jnp.concatenate([W_g1, W_g1 @ blk_s, W_g1 @ blk_d], axis=1)
    f1 = _mm(y, wcat1, jnp.zeros((wcat1.shape[1],), jnp.float32))
    xp1 = f1[:, :h1dim].reshape(n, heads1, ch1)
    out1 = _gat_aggregate(xp1, f1[:, h1dim:h1dim + heads1],
                          f1[:, h1dim + heads1:], s, d, n)
    h1 = jnp.maximum(out1.reshape(n, h1dim) + b_g1, 0.0)

    h2dim = W_g2.shape[1]
    wcat2 = jnp.concatenate([W_g2, W_g2 @ a2_s[0][:, None],
                             W_g2 @ a2_d[0][:, None]], axis=1)
    f2 = _mm(h1, wcat2, jnp.zeros((wcat2.shape[1],), jnp.float32))
    xp2 = f2[:, :h2dim].reshape(n, 1, h2dim)
    out2 = _gat_aggregate(xp2, f2[:, h2dim:h2dim + 1],
                          f2[:, h2dim + 1:], s, d, n)
    bl1 = b_l1 + b_g2 @ W_l1
    tile_m = 1000
    h = pl.pallas_call(
        _mlp_kernel,
        grid=(n // tile_m,),
        in_specs=[
            pl.BlockSpec((tile_m, h2dim), lambda i: (i, 0)),
            pl.BlockSpec((h2dim, h2dim), lambda i: (0, 0)),
            pl.BlockSpec((1, h2dim), lambda i: (0, 0)),
            pl.BlockSpec((h2dim, h2dim), lambda i: (0, 0)),
            pl.BlockSpec((1, h2dim), lambda i: (0, 0)),
        ],
        out_specs=pl.BlockSpec((tile_m, h2dim), lambda i: (i, 0)),
        out_shape=jax.ShapeDtypeStruct((n, h2dim), jnp.float32),
    )(out2.reshape(n, h2dim), W_l1, bl1.reshape(1, h2dim),
      W_l2, b_l2.reshape(1, h2dim))

    node_id = edge_index[:, train_edge_id]
    ha = h[node_id[0]]
    hb = h[node_id[1]]
    ne = ha.shape[0]
    cls = W_fc2.shape[1]
    tile_e = 1024
    out = pl.pallas_call(
        _fuse_kernel,
        grid=(ne // tile_e,),
        in_specs=[
            pl.BlockSpec((tile_e, h2dim), lambda i: (i, 0)),
            pl.BlockSpec((tile_e, h2dim), lambda i: (i, 0)),
            pl.BlockSpec((h2dim, cls), lambda i: (0, 0)),
            pl.BlockSpec((1, cls), lambda i: (0, 0)),
        ],
        out_specs=pl.BlockSpec((tile_e, cls), lambda i: (i, 0)),
        out_shape=jax.ShapeDtypeStruct((ne, cls), jnp.float32),
    )(ha, hb, W_fc2, b_fc2.reshape(1, cls))
    return out
